# baseline (speedup 1.0000x reference)
import functools

import jax
import jax.numpy as jnp
from jax import lax
from jax.experimental import pallas as pl
from jax.experimental.pallas import tpu as pltpu
from jax.experimental.pallas import tpu_sc as plsc

_B = 4096
_L = 50
_NFEAT = 64
_EMB = 128
_NC, _NS = 2, 16
_NW = _NC * _NS
_CHUNK = 2
_IDXJ = _CHUNK * _L
_LANES = 16
_NK = _EMB // _LANES
_NBUF = 8
_SPLIT = 1


def _acc_step(buf_v, out_v, step):
    for half in range(_CHUNK):
        base = half * _L

        def body(r, accs):
            return tuple(a + buf_v[r, pl.ds(_LANES * k, _LANES)]
                         for k, a in enumerate(accs))

        init = tuple(buf_v[base, pl.ds(_LANES * k, _LANES)]
                     for k in range(_NK))
        accs = lax.fori_loop(base + 1, base + _L, body, init)
        row = step * _CHUNK + half
        for k in range(_NK):
            out_v[row, pl.ds(_LANES * k, _LANES)] = accs[k]


def _make_bag_body(bpw):
    nj = bpw // _CHUNK

    def _bag_body(table_hbm, idx_hbm, out_hbm, *scratch):
        idx_v = scratch[0]
        bufs = scratch[1:1 + _NBUF]
        out_v = scratch[1 + _NBUF]
        sems = scratch[2 + _NBUF:]
        wid = lax.axis_index("s") * _NC + lax.axis_index("c")
        pltpu.sync_copy(idx_hbm.at[wid], idx_v)

        def start(j, b):
            pltpu.async_copy(table_hbm.at[idx_v.at[j]], bufs[b], sems[b])

        def wait(b):
            pltpu.make_async_copy(table_hbm.at[idx_v.at[0]], bufs[b],
                                  sems[b]).wait()

        for b in range(_NBUF - 1):
            start(b, b)

        @pl.loop(0, nj, step=_NBUF)
        def _(j):
            for b in range(_NBUF):
                nxt = j + b + _NBUF - 1

                @pl.when(nxt < nj)
                def _():
                    start(nxt, (b + _NBUF - 1) % _NBUF)

                wait(b)
                _acc_step(bufs[b], out_v, j + b)

        pltpu.sync_copy(out_v, out_hbm.at[pl.ds(wid * bpw, bpw)])

    return _bag_body


def _emb_bag(table, idx_grouped, nrows):
    bpw = nrows // _NW
    nj = bpw // _CHUNK
    mesh = plsc.VectorSubcoreMesh(core_axis_name="c", subcore_axis_name="s")
    kfn = pl.kernel(
        _make_bag_body(bpw),
        out_type=jax.ShapeDtypeStruct((nrows, _EMB), jnp.float32),
        mesh=mesh,
        scratch_types=(
            [pltpu.VMEM((nj, _IDXJ), jnp.int32)]
            + [pltpu.VMEM((_IDXJ, _EMB), jnp.float32)] * _NBUF
            + [pltpu.VMEM((bpw, _EMB), jnp.float32)]
            + [pltpu.SemaphoreType.DMA] * _NBUF
        ),
    )
    return kfn(table, idx_grouped)


def _mlp_body(x_ref, e_ref, w1_ref, b1_ref, w2_ref, b2_ref,
              w3_ref, b3_ref, o_ref):
    w1 = w1_ref[...]
    h1 = jnp.dot(x_ref[...], w1[:_NFEAT], preferred_element_type=jnp.float32)
    h1 += jnp.dot(e_ref[...], w1[_NFEAT:], preferred_element_type=jnp.float32)
    h1 = jnp.maximum(h1 + b1_ref[...], 0.0)
    h2 = jnp.maximum(
        jnp.dot(h1, w2_ref[...], preferred_element_type=jnp.float32)
        + b2_ref[...], 0.0)
    o_ref[...] = (jnp.dot(h2, w3_ref[...], preferred_element_type=jnp.float32)
                  + b3_ref[...])


def _mlp(x_num, emb, W1, b1, W2, b2, W3, b3):
    b = x_num.shape[0]
    bm = b
    grid = (b // bm,)
    h1, h2, out = W1.shape[1], W2.shape[1], W3.shape[1]
    full = lambda shape: pl.BlockSpec(shape, lambda i: (0,) * len(shape))
    return pl.pallas_call(
        _mlp_body,
        grid=grid,
        in_specs=[
            pl.BlockSpec((bm, _NFEAT), lambda i: (i, 0)),
            pl.BlockSpec((bm, _EMB), lambda i: (i, 0)),
            full((_NFEAT + _EMB, h1)),
            full((h1,)),
            full((h1, h2)),
            full((h2,)),
            full((h2, out)),
            full((out,)),
        ],
        out_specs=pl.BlockSpec((bm, out), lambda i: (i, 0)),
        out_shape=jax.ShapeDtypeStruct((b, out), jnp.float32),
    )(x_num, emb, W1, b1, W2, b2, W3, b3)


def kernel(x_num, leaf_ids, table, W1, b1, W2, b2, W3, b3):
    nj = _B // _NW // _CHUNK
    idx = leaf_ids.astype(jnp.int32).reshape(_NW, nj, _IDXJ)
    emb = _emb_bag(table, idx, _B)
    return _mlp(x_num, emb, W1, b1, W2, b2, W3, b3)


# speedup vs baseline: 1.0000x; 1.0000x over previous
import functools

import jax
import jax.numpy as jnp
from jax import lax
from jax.experimental import pallas as pl
from jax.experimental.pallas import tpu as pltpu
from jax.experimental.pallas import tpu_sc as plsc

_B = 4096
_L = 50
_NFEAT = 64
_EMB = 128
_NC, _NS = 2, 16
_NW = _NC * _NS
_CHUNK = 2
_IDXJ = _CHUNK * _L
_LANES = 16
_NK = _EMB // _LANES
_NBUF = 8
_SPLIT = 1


def _acc_step(buf_v, out_v, step):
    for half in range(_CHUNK):
        base = half * _L

        def body(r, accs):
            return tuple(a + buf_v[r, pl.ds(_LANES * k, _LANES)]
                         for k, a in enumerate(accs))

        init = tuple(buf_v[base, pl.ds(_LANES * k, _LANES)]
                     for k in range(_NK))
        accs = lax.fori_loop(base + 1, base + _L, body, init)
        row = step * _CHUNK + half
        for k in range(_NK):
            out_v[row, pl.ds(_LANES * k, _LANES)] = accs[k]


def _make_bag_body(bpw):
    nj = bpw // _CHUNK

    def _bag_body(table_hbm, idx_hbm, out_hbm, *scratch):
        idx_v = scratch[0]
        bufs = scratch[1:1 + _NBUF]
        out_v = scratch[1 + _NBUF]
        sems = scratch[2 + _NBUF:]
        wid = lax.axis_index("s") * _NC + lax.axis_index("c")
        pltpu.sync_copy(idx_hbm.at[wid], idx_v)

        def start(j, b):
            pltpu.async_copy(table_hbm.at[idx_v.at[j]], bufs[b], sems[b])

        def wait(b):
            pltpu.make_async_copy(table_hbm.at[idx_v.at[0]], bufs[b],
                                  sems[b]).wait()

        for b in range(_NBUF - 1):
            start(b, b)

        @pl.loop(0, nj, step=_NBUF)
        def _(j):
            for b in range(_NBUF):
                nxt = j + b + _NBUF - 1

                @pl.when(nxt < nj)
                def _():
                    start(nxt, (b + _NBUF - 1) % _NBUF)

                wait(b)
                _acc_step(bufs[b], out_v, j + b)

        pltpu.sync_copy(out_v, out_hbm.at[pl.ds(wid * bpw, bpw)])

    return _bag_body


def _emb_bag(table, idx_grouped, nrows):
    bpw = nrows // _NW
    nj = bpw // _CHUNK
    mesh = plsc.VectorSubcoreMesh(core_axis_name="c", subcore_axis_name="s")
    kfn = pl.kernel(
        _make_bag_body(bpw),
        out_type=jax.ShapeDtypeStruct((nrows, _EMB), jnp.float32),
        mesh=mesh,
        scratch_types=(
            [pltpu.VMEM((nj, _IDXJ), jnp.int32)]
            + [pltpu.VMEM((_IDXJ, _EMB), jnp.float32)] * _NBUF
            + [pltpu.VMEM((bpw, _EMB), jnp.float32)]
            + [pltpu.SemaphoreType.DMA] * _NBUF
        ),
    )
    return kfn(table, idx_grouped)


def _mlp_body(x_ref, e_ref, w1_ref, b1_ref, w2_ref, b2_ref,
              w3_ref, b3_ref, o_ref):
    w1 = w1_ref[...]
    h1 = jnp.dot(x_ref[...], w1[:_NFEAT], preferred_element_type=jnp.float32)
    h1 += jnp.dot(e_ref[...], w1[_NFEAT:], preferred_element_type=jnp.float32)
    h1 = jnp.maximum(h1 + b1_ref[...], 0.0)
    h2 = jnp.maximum(
        jnp.dot(h1, w2_ref[...], preferred_element_type=jnp.float32)
        + b2_ref[...], 0.0)
    o_ref[...] = (jnp.dot(h2, w3_ref[...], preferred_element_type=jnp.float32)
                  + b3_ref[...])


def _mlp(x_num, emb, W1, b1, W2, b2, W3, b3):
    b = x_num.shape[0]
    bm = 1024
    grid = (b // bm,)
    h1, h2, out = W1.shape[1], W2.shape[1], W3.shape[1]
    full = lambda shape: pl.BlockSpec(shape, lambda i: (0,) * len(shape))
    return pl.pallas_call(
        _mlp_body,
        grid=grid,
        in_specs=[
            pl.BlockSpec((bm, _NFEAT), lambda i: (i, 0)),
            pl.BlockSpec((bm, _EMB), lambda i: (i, 0)),
            full((_NFEAT + _EMB, h1)),
            full((h1,)),
            full((h1, h2)),
            full((h2,)),
            full((h2, out)),
            full((out,)),
        ],
        out_specs=pl.BlockSpec((bm, out), lambda i: (i, 0)),
        out_shape=jax.ShapeDtypeStruct((b, out), jnp.float32),
    )(x_num, emb, W1, b1, W2, b2, W3, b3)


def kernel(x_num, leaf_ids, table, W1, b1, W2, b2, W3, b3):
    nj = _B // _NW // _CHUNK
    idx = leaf_ids.astype(jnp.int32).reshape(_NW, nj, _IDXJ)
    emb = _emb_bag(table, idx, _B)
    return _mlp(x_num, emb, W1, b1, W2, b2, W3, b3)


# speedup vs baseline: 1.0092x; 1.0092x over previous
import functools

import jax
import jax.numpy as jnp
from jax import lax
from jax.experimental import pallas as pl
from jax.experimental.pallas import tpu as pltpu
from jax.experimental.pallas import tpu_sc as plsc

_B = 4096
_L = 50
_NFEAT = 64
_EMB = 128
_NC, _NS = 2, 16
_NW = _NC * _NS
_CHUNK = 2
_IDXJ = _CHUNK * _L
_LANES = 16
_NK = _EMB // _LANES
_NBUF = 8
_SPLIT = 1


def _acc_step(buf_v, out_v, step):
    for half in range(_CHUNK):
        base = half * _L

        def body(r, accs):
            return tuple(a + buf_v[r, pl.ds(_LANES * k, _LANES)]
                         for k, a in enumerate(accs))

        init = tuple(buf_v[base, pl.ds(_LANES * k, _LANES)]
                     for k in range(_NK))
        accs = lax.fori_loop(base + 1, base + _L, body, init)
        row = step * _CHUNK + half
        for k in range(_NK):
            out_v[row, pl.ds(_LANES * k, _LANES)] = accs[k]


def _make_bag_body(bpw):
    nj = bpw // _CHUNK

    def _bag_body(table_hbm, idx_hbm, out_hbm, *scratch):
        idx_v = scratch[0]
        bufs = scratch[1:1 + _NBUF]
        out_v = scratch[1 + _NBUF]
        sems = scratch[2 + _NBUF:]
        wid = lax.axis_index("s") * _NC + lax.axis_index("c")
        pltpu.sync_copy(idx_hbm.at[wid], idx_v)

        def start(j, b):
            pltpu.async_copy(table_hbm.at[idx_v.at[j]], bufs[b], sems[b])

        def wait(b):
            pltpu.make_async_copy(table_hbm.at[idx_v.at[0]], bufs[b],
                                  sems[b]).wait()

        for b in range(_NBUF - 1):
            start(b, b)

        @pl.loop(0, nj, step=_NBUF)
        def _(j):
            for b in range(_NBUF):
                nxt = j + b + _NBUF - 1

                @pl.when(nxt < nj)
                def _():
                    start(nxt, (b + _NBUF - 1) % _NBUF)

                wait(b)
                _acc_step(bufs[b], out_v, j + b)

        pltpu.sync_copy(out_v, out_hbm.at[pl.ds(wid * bpw, bpw)])

    return _bag_body


def _emb_bag(table, idx_grouped, nrows):
    bpw = nrows // _NW
    nj = bpw // _CHUNK
    mesh = plsc.VectorSubcoreMesh(core_axis_name="c", subcore_axis_name="s")
    kfn = pl.kernel(
        _make_bag_body(bpw),
        out_type=jax.ShapeDtypeStruct((nrows, _EMB), jnp.float32),
        mesh=mesh,
        scratch_types=(
            [pltpu.VMEM((nj, _IDXJ), jnp.int32)]
            + [pltpu.VMEM((_IDXJ, _EMB), jnp.float32)] * _NBUF
            + [pltpu.VMEM((bpw, _EMB), jnp.float32)]
            + [pltpu.SemaphoreType.DMA] * _NBUF
        ),
    )
    return kfn(table, idx_grouped)


def _mlp_body(x_ref, e_ref, w1_ref, b1_ref, w2_ref, b2_ref,
              w3_ref, b3_ref, o_ref):
    bf = jnp.bfloat16
    w1 = w1_ref[...].astype(bf)
    h1 = jnp.dot(x_ref[...].astype(bf), w1[:_NFEAT],
                 preferred_element_type=jnp.float32)
    h1 += jnp.dot(e_ref[...].astype(bf), w1[_NFEAT:],
                  preferred_element_type=jnp.float32)
    h1 = jnp.maximum(h1 + b1_ref[...], 0.0)
    h2 = jnp.maximum(
        jnp.dot(h1.astype(bf), w2_ref[...].astype(bf),
                preferred_element_type=jnp.float32)
        + b2_ref[...], 0.0)
    o_ref[...] = (jnp.dot(h2, w3_ref[...], preferred_element_type=jnp.float32)
                  + b3_ref[...])


def _mlp(x_num, emb, W1, b1, W2, b2, W3, b3):
    b = x_num.shape[0]
    bm = 2048
    grid = (b // bm,)
    h1, h2, out = W1.shape[1], W2.shape[1], W3.shape[1]
    full = lambda shape: pl.BlockSpec(shape, lambda i: (0,) * len(shape))
    return pl.pallas_call(
        _mlp_body,
        grid=grid,
        in_specs=[
            pl.BlockSpec((bm, _NFEAT), lambda i: (i, 0)),
            pl.BlockSpec((bm, _EMB), lambda i: (i, 0)),
            full((_NFEAT + _EMB, h1)),
            full((h1,)),
            full((h1, h2)),
            full((h2,)),
            full((h2, out)),
            full((out,)),
        ],
        out_specs=pl.BlockSpec((bm, out), lambda i: (i, 0)),
        out_shape=jax.ShapeDtypeStruct((b, out), jnp.float32),
    )(x_num, emb, W1, b1, W2, b2, W3, b3)


def kernel(x_num, leaf_ids, table, W1, b1, W2, b2, W3, b3):
    nj = _B // _NW // _CHUNK
    idx = leaf_ids.astype(jnp.int32).reshape(_NW, nj, _IDXJ)
    emb = _emb_bag(table, idx, _B)
    return _mlp(x_num, emb, W1, b1, W2, b2, W3, b3)
